# Initial kernel scaffold; baseline (speedup 1.0000x reference)
import jax
import jax.numpy as jnp
from jax.experimental import pallas as pl


def kernel(week_numbers, week_embed):
    raise NotImplementedError("write your pallas kernel here")


# speedup vs baseline: 2.9048x; 2.9048x over previous
import functools

import jax
import jax.numpy as jnp
from jax import lax
from jax.experimental import pallas as pl
from jax.experimental.pallas import tpu as pltpu
from jax.experimental.pallas import tpu_sc as plsc

MAX_WEEKS = 160
EMBED_DIM = 64
BATCH = 16384
HIST = 200

N = BATCH * HIST
NC, NS = 2, 16
NW = NC * NS
PER_W = N // NW
IDX_MINOR = 128
CHUNK = 1024
ROWS_PER = CHUNK // IDX_MINOR
N_ITER = PER_W // CHUNK

_mesh = plsc.VectorSubcoreMesh(core_axis_name="c", subcore_axis_name="s")


@functools.partial(
    pl.kernel,
    out_type=jax.ShapeDtypeStruct((N, EMBED_DIM), jnp.float32),
    mesh=_mesh,
    scratch_types=[
        pltpu.VMEM((ROWS_PER, IDX_MINOR), jnp.int32),
        pltpu.VMEM((CHUNK, EMBED_DIM), jnp.float32),
        pltpu.SemaphoreType.DMA,
    ],
    compiler_params=pltpu.CompilerParams(use_tc_tiling_on_sc=False),
)
def _gather_kernel(idx_hbm, table_hbm, out_hbm, idx_v, rows_v, sem):
    wid = lax.axis_index("s") * NC + lax.axis_index("c")
    row0 = wid * (PER_W // IDX_MINOR)

    def step(t, carry):
        irow = row0 + t * ROWS_PER
        pltpu.sync_copy(idx_hbm.at[pl.ds(irow, ROWS_PER)], idx_v)
        handles = [
            pltpu.async_copy(
                table_hbm.at[idx_v.at[j]],
                rows_v.at[pl.ds(j * IDX_MINOR, IDX_MINOR)],
                sem,
            )
            for j in range(ROWS_PER)
        ]
        for h in handles:
            h.wait()
        out0 = wid * PER_W + t * CHUNK
        pltpu.sync_copy(rows_v, out_hbm.at[pl.ds(out0, CHUNK)])
        return carry

    lax.fori_loop(0, N_ITER, step, 0)


def kernel(week_numbers, week_embed):
    idx = week_numbers.reshape(N).astype(jnp.int32).reshape(N // IDX_MINOR, IDX_MINOR)
    out = _gather_kernel(idx, week_embed)
    return out.reshape(BATCH, HIST, EMBED_DIM)


# speedup vs baseline: 5.8258x; 2.0056x over previous
import functools

import jax
import jax.numpy as jnp
from jax import lax
from jax.experimental import pallas as pl
from jax.experimental.pallas import tpu as pltpu
from jax.experimental.pallas import tpu_sc as plsc

MAX_WEEKS = 160
EMBED_DIM = 64
BATCH = 16384
HIST = 200

N = BATCH * HIST
NC, NS = 2, 16
NW = NC * NS
PER_W = N // NW
IDX_MINOR = 128
CHUNK = 512
ROWS_PER = CHUNK // IDX_MINOR
N_ITER = PER_W // CHUNK
NBUF = 2
N_OUTER = N_ITER // NBUF

_mesh = plsc.VectorSubcoreMesh(core_axis_name="c", subcore_axis_name="s")


@functools.partial(
    pl.kernel,
    out_type=jax.ShapeDtypeStruct((N, EMBED_DIM), jnp.float32),
    mesh=_mesh,
    scratch_types=[
        pltpu.VMEM((NBUF, ROWS_PER, IDX_MINOR), jnp.int32),
        pltpu.VMEM((NBUF, CHUNK, EMBED_DIM), jnp.float32),
        pltpu.VMEM_SHARED((MAX_WEEKS, EMBED_DIM), jnp.float32),
        pltpu.SemaphoreType.DMA,
        pltpu.SemaphoreType.DMA,
        pltpu.SemaphoreType.DMA,
        pltpu.SemaphoreType.DMA,
    ],
    compiler_params=pltpu.CompilerParams(use_tc_tiling_on_sc=False),
)
def _gather_kernel(idx_hbm, table_hbm, out_hbm, idx_v, rows_v, table_v,
                   isem, gsem, osem_a, osem_b):
    wid = lax.axis_index("s") * NC + lax.axis_index("c")

    @pl.when(lax.axis_index("s") == 0)
    def _stage_table():
        pltpu.sync_copy(table_hbm, table_v)

    plsc.subcore_barrier()

    base_irow = wid * (PER_W // IDX_MINOR)
    base_out = wid * PER_W
    osems = [osem_a, osem_b]

    pltpu.async_copy(idx_hbm.at[pl.ds(base_irow, ROWS_PER)], idx_v.at[0], isem)

    def outer(o, carry):
        for b in range(NBUF):
            t = NBUF * o + b
            pltpu.make_async_copy(
                idx_hbm.at[pl.ds(0, ROWS_PER)], idx_v.at[b], isem
            ).wait()

            @pl.when(t + 1 < N_ITER)
            def _prefetch():
                irow = base_irow + (t + 1) * ROWS_PER
                pltpu.async_copy(
                    idx_hbm.at[pl.ds(irow, ROWS_PER)], idx_v.at[1 - b], isem
                )

            @pl.when(t >= NBUF)
            def _drain_prev_out():
                pltpu.make_async_copy(
                    rows_v.at[b], out_hbm.at[pl.ds(0, CHUNK)], osems[b]
                ).wait()

            handles = [
                pltpu.async_copy(
                    table_v.at[idx_v.at[b].at[j]],
                    rows_v.at[b].at[pl.ds(j * IDX_MINOR, IDX_MINOR)],
                    gsem,
                )
                for j in range(ROWS_PER)
            ]
            for h in handles:
                h.wait()

            pltpu.async_copy(
                rows_v.at[b],
                out_hbm.at[pl.ds(base_out + t * CHUNK, CHUNK)],
                osems[b],
            )
        return carry

    lax.fori_loop(0, N_OUTER, outer, 0)

    for b in range(NBUF):
        pltpu.make_async_copy(
            rows_v.at[b], out_hbm.at[pl.ds(0, CHUNK)], osems[b]
        ).wait()


def kernel(week_numbers, week_embed):
    idx = week_numbers.reshape(N).astype(jnp.int32).reshape(N // IDX_MINOR, IDX_MINOR)
    out = _gather_kernel(idx, week_embed)
    return out.reshape(BATCH, HIST, EMBED_DIM)
